# baseline (speedup 1.0000x reference)
import functools

import jax
import jax.numpy as jnp
from jax import lax
from jax.experimental import pallas as pl
from jax.experimental.pallas import tpu as pltpu
from jax.experimental.pallas import tpu_sc as plsc

B0, B1, DIM = 16384, 200, 32
NC, NS = 2, 16
NW = NC * NS
TPW = B0 // NW
LPW = TPW // 128
SI_D = DIM // 8
T_HALF = B1 // 2

_mesh = plsc.VectorSubcoreMesh(core_axis_name="c", subcore_axis_name="s")


@functools.partial(
    pl.kernel,
    mesh=_mesh,
    compiler_params=pltpu.CompilerParams(
        use_tc_tiling_on_sc=False, needs_layout_passes=False
    ),
    out_type=jax.ShapeDtypeStruct((B1, SI_D, 128, 8, 128), jnp.float32),
    scratch_types=[
        pltpu.VMEM((2, LPW, 128), jnp.int32),
        pltpu.VMEM((2, TPW, DIM), jnp.float32),
        pltpu.VMEM((2, SI_D, LPW, 8, 128), jnp.float32),
        pltpu.SemaphoreType.DMA,
        pltpu.SemaphoreType.DMA,
        pltpu.SemaphoreType.DMA,
    ],
)
def _emb_lookup(tok5, table, out5, idx_v, rows_v, plane_v, sem_g0, sem_g1, sem_o):
    wid = lax.axis_index("s") * NC + lax.axis_index("c")
    li0 = wid * LPW
    iota = lax.iota(jnp.int32, 16)

    def load_idx(b1, slot):
        pltpu.sync_copy(
            tok5.at[b1 // 8, pl.ds(li0, LPW), lax.rem(b1, 8)], idx_v.at[slot]
        )

    def fire_gathers(slot, sem):
        for q in range(LPW):
            pltpu.async_copy(
                table.at[idx_v.at[slot, q]],
                rows_v.at[slot, pl.ds(q * 128, 128)],
                sem,
            )

    def drain_gathers(slot, sem):
        for q in range(LPW):
            pltpu.make_async_copy(
                table.at[idx_v.at[slot, 0]],
                rows_v.at[slot, pl.ds(0, 128)],
                sem,
            ).wait()

    def drain_writebacks(n):
        for _ in range(n):
            pltpu.make_async_copy(
                plane_v.at[0, 0], out5.at[0, 0, pl.ds(0, LPW)], sem_o
            ).wait()

    def transpose_chunk(slot):
        rows = rows_v.at[slot]

        def dloop(d, carry):
            si = d // 8
            dr = lax.rem(d, 8)
            dvec = jnp.full((16,), 0, jnp.int32) + d
            for li in range(LPW):
                for br0 in range(8):
                    rvec = iota + (li * 128 + br0 * 16)
                    vec = plsc.load_gather(rows, [rvec, dvec])
                    plane_v[slot, si, li, dr, pl.ds(br0 * 16, 16)] = vec
            return carry

        lax.fori_loop(0, DIM, dloop, 0)

    def fire_writebacks(slot, b1):
        for si in range(SI_D):
            pltpu.async_copy(
                plane_v.at[slot, si],
                out5.at[b1, si, pl.ds(li0, LPW)],
                sem_o,
            )

    load_idx(0, 0)
    fire_gathers(0, sem_g0)

    def body(t, carry):
        b1a = 2 * t

        load_idx(b1a + 1, 1)
        fire_gathers(1, sem_g1)

        @pl.when(t >= 1)
        def _():
            drain_writebacks(SI_D)

        drain_gathers(0, sem_g0)
        transpose_chunk(0)
        fire_writebacks(0, b1a)

        @pl.when(t < T_HALF - 1)
        def _():
            load_idx(b1a + 2, 0)
            fire_gathers(0, sem_g0)

        @pl.when(t >= 1)
        def _():
            drain_writebacks(SI_D)

        drain_gathers(1, sem_g1)
        transpose_chunk(1)
        fire_writebacks(1, b1a + 1)
        return carry

    lax.fori_loop(0, T_HALF, body, 0)
    drain_writebacks(2 * SI_D)


def kernel(token_ids, weight):
    tok5 = (
        jnp.asarray(token_ids, jnp.int32)
        .reshape(128, 128, B1 // 8, 8)
        .transpose(2, 0, 3, 1)
    )
    out5 = _emb_lookup(tok5, weight)
    return out5.transpose(2, 4, 0, 1, 3).reshape(B0, B1, DIM)


# speedup vs baseline: 1.6002x; 1.6002x over previous
import functools

import jax
import jax.numpy as jnp
from jax import lax
from jax.experimental import pallas as pl
from jax.experimental.pallas import tpu as pltpu
from jax.experimental.pallas import tpu_sc as plsc

B0, B1, DIM = 16384, 200, 32
NC, NS = 2, 16
NW = NC * NS
TPW = B0 // NW
LPW = TPW // 128
SI_D = DIM // 8
T_HALF = B1 // 2

_mesh = plsc.VectorSubcoreMesh(core_axis_name="c", subcore_axis_name="s")


@functools.partial(
    pl.kernel,
    mesh=_mesh,
    compiler_params=pltpu.CompilerParams(
        use_tc_tiling_on_sc=False, needs_layout_passes=False
    ),
    out_type=jax.ShapeDtypeStruct((B1, SI_D, 128, 8, 128), jnp.float32),
    scratch_types=[
        pltpu.VMEM((2, LPW, 128), jnp.int32),
        pltpu.VMEM((2, TPW, DIM), jnp.float32),
        pltpu.VMEM((2, SI_D, LPW, 8, 128), jnp.float32),
        pltpu.SemaphoreType.DMA,
        pltpu.SemaphoreType.DMA,
        pltpu.SemaphoreType.DMA,
    ],
)
def _emb_lookup(tok5, table, out5, idx_v, rows_v, plane_v, sem_g0, sem_g1, sem_o):
    wid = lax.axis_index("s") * NC + lax.axis_index("c")
    li0 = wid * LPW
    iota = lax.iota(jnp.int32, 16)

    def load_idx(b1, slot):
        pltpu.sync_copy(
            tok5.at[b1 // 8, pl.ds(li0, LPW), lax.rem(b1, 8)], idx_v.at[slot]
        )

    def fire_gathers(slot, sem):
        for q in range(LPW):
            pltpu.async_copy(
                table.at[idx_v.at[slot, q]],
                rows_v.at[slot, pl.ds(q * 128, 128)],
                sem,
            )

    def drain_gathers(slot, sem):
        for q in range(LPW):
            pltpu.make_async_copy(
                table.at[idx_v.at[slot, 0]],
                rows_v.at[slot, pl.ds(0, 128)],
                sem,
            ).wait()

    def drain_writebacks(n):
        for _ in range(n):
            pltpu.make_async_copy(
                plane_v.at[0, 0], out5.at[0, 0, pl.ds(0, LPW)], sem_o
            ).wait()

    rvecs = [
        iota + (li * 128 + br0 * 16) for li in range(LPW) for br0 in range(8)
    ]

    def transpose_chunk(slot):
        rows = rows_v.at[slot]

        @plsc.parallel_loop(0, DIM, step=1, unroll=2)
        def dloop(d):
            si = d // 8
            dr = lax.rem(d, 8)
            dvec = jnp.full((16,), 0, jnp.int32) + d
            for li in range(LPW):
                for br0 in range(8):
                    vec = plsc.load_gather(rows, [rvecs[li * 8 + br0], dvec])
                    plane_v[slot, si, li, dr, pl.ds(br0 * 16, 16)] = vec

    def fire_writebacks(slot, b1):
        for si in range(SI_D):
            pltpu.async_copy(
                plane_v.at[slot, si],
                out5.at[b1, si, pl.ds(li0, LPW)],
                sem_o,
            )

    load_idx(0, 0)
    fire_gathers(0, sem_g0)

    def body(t, carry):
        b1a = 2 * t

        load_idx(b1a + 1, 1)
        fire_gathers(1, sem_g1)

        @pl.when(t >= 1)
        def _():
            drain_writebacks(SI_D)

        drain_gathers(0, sem_g0)
        transpose_chunk(0)
        fire_writebacks(0, b1a)

        @pl.when(t < T_HALF - 1)
        def _():
            load_idx(b1a + 2, 0)
            fire_gathers(0, sem_g0)

        @pl.when(t >= 1)
        def _():
            drain_writebacks(SI_D)

        drain_gathers(1, sem_g1)
        transpose_chunk(1)
        fire_writebacks(1, b1a + 1)
        return carry

    lax.fori_loop(0, T_HALF, body, 0)
    drain_writebacks(2 * SI_D)


def kernel(token_ids, weight):
    tok5 = (
        jnp.asarray(token_ids, jnp.int32)
        .reshape(128, 128, B1 // 8, 8)
        .transpose(2, 0, 3, 1)
    )
    out5 = _emb_lookup(tok5, weight)
    return out5.transpose(2, 4, 0, 1, 3).reshape(B0, B1, DIM)


# speedup vs baseline: 1.6016x; 1.0009x over previous
import functools

import jax
import jax.numpy as jnp
from jax import lax
from jax.experimental import pallas as pl
from jax.experimental.pallas import tpu as pltpu
from jax.experimental.pallas import tpu_sc as plsc

B0, B1, DIM = 16384, 200, 32
NC, NS = 2, 16
NW = NC * NS
TPW = B0 // NW
LPW = TPW // 128
SI_D = DIM // 8
T_HALF = B1 // 2

_mesh = plsc.VectorSubcoreMesh(core_axis_name="c", subcore_axis_name="s")


@functools.partial(
    pl.kernel,
    mesh=_mesh,
    compiler_params=pltpu.CompilerParams(
        use_tc_tiling_on_sc=False, needs_layout_passes=False
    ),
    out_type=jax.ShapeDtypeStruct((B1, SI_D, 128, 8, 128), jnp.float32),
    scratch_types=[
        pltpu.VMEM((2, LPW, 128), jnp.int32),
        pltpu.VMEM((2, TPW, DIM), jnp.float32),
        pltpu.VMEM((2, SI_D, LPW, 8, 128), jnp.float32),
        pltpu.SemaphoreType.DMA,
        pltpu.SemaphoreType.DMA,
        pltpu.SemaphoreType.DMA,
    ],
)
def _emb_lookup(tok5, table, out5, idx_v, rows_v, plane_v, sem_g0, sem_g1, sem_o):
    wid = lax.axis_index("s") * NC + lax.axis_index("c")
    li0 = wid * LPW
    iota = lax.iota(jnp.int32, 16)

    def load_idx(b1, slot):
        pltpu.sync_copy(
            tok5.at[b1 // 8, pl.ds(li0, LPW), lax.rem(b1, 8)], idx_v.at[slot]
        )

    def fire_gathers(slot, sem):
        for q in range(LPW):
            pltpu.async_copy(
                table.at[idx_v.at[slot, q]],
                rows_v.at[slot, pl.ds(q * 128, 128)],
                sem,
            )

    def drain_gathers(slot, sem):
        for q in range(LPW):
            pltpu.make_async_copy(
                table.at[idx_v.at[slot, 0]],
                rows_v.at[slot, pl.ds(0, 128)],
                sem,
            ).wait()

    def drain_writebacks(n):
        for _ in range(n):
            pltpu.make_async_copy(
                plane_v.at[0, 0], out5.at[0, 0, pl.ds(0, LPW)], sem_o
            ).wait()

    rvecs = [
        iota + (li * 128 + br0 * 16) for li in range(LPW) for br0 in range(8)
    ]

    def transpose_chunk(slot):
        rows = rows_v.at[slot]

        @plsc.parallel_loop(0, DIM, step=1, unroll=4)
        def dloop(d):
            si = d // 8
            dr = lax.rem(d, 8)
            dvec = jnp.full((16,), 0, jnp.int32) + d
            for li in range(LPW):
                for br0 in range(8):
                    vec = plsc.load_gather(rows, [rvecs[li * 8 + br0], dvec])
                    plane_v[slot, si, li, dr, pl.ds(br0 * 16, 16)] = vec

    def fire_writebacks(slot, b1):
        for si in range(SI_D):
            pltpu.async_copy(
                plane_v.at[slot, si],
                out5.at[b1, si, pl.ds(li0, LPW)],
                sem_o,
            )

    load_idx(0, 0)
    fire_gathers(0, sem_g0)

    def body(t, carry):
        b1a = 2 * t

        load_idx(b1a + 1, 1)
        fire_gathers(1, sem_g1)

        @pl.when(t >= 1)
        def _():
            drain_writebacks(SI_D)

        drain_gathers(0, sem_g0)
        transpose_chunk(0)
        fire_writebacks(0, b1a)

        @pl.when(t < T_HALF - 1)
        def _():
            load_idx(b1a + 2, 0)
            fire_gathers(0, sem_g0)

        @pl.when(t >= 1)
        def _():
            drain_writebacks(SI_D)

        drain_gathers(1, sem_g1)
        transpose_chunk(1)
        fire_writebacks(1, b1a + 1)
        return carry

    lax.fori_loop(0, T_HALF, body, 0)
    drain_writebacks(2 * SI_D)


def kernel(token_ids, weight):
    tok5 = (
        jnp.asarray(token_ids, jnp.int32)
        .reshape(128, 128, B1 // 8, 8)
        .transpose(2, 0, 3, 1)
    )
    out5 = _emb_lookup(tok5, weight)
    return out5.transpose(2, 4, 0, 1, 3).reshape(B0, B1, DIM)


# speedup vs baseline: 3.2231x; 2.0124x over previous
import functools

import jax
import jax.numpy as jnp
from jax import lax
from jax.experimental import pallas as pl
from jax.experimental.pallas import tpu as pltpu
from jax.experimental.pallas import tpu_sc as plsc

B0, B1, DIM = 16384, 200, 32
NC, NS = 2, 16
NW = NC * NS
TPW = B0 // NW
LPW = TPW // 128
SI_D = DIM // 8
T_HALF = B1 // 2

_mesh = plsc.VectorSubcoreMesh(core_axis_name="c", subcore_axis_name="s")


@functools.partial(
    pl.kernel,
    mesh=_mesh,
    compiler_params=pltpu.CompilerParams(
        use_tc_tiling_on_sc=False, needs_layout_passes=False
    ),
    out_type=jax.ShapeDtypeStruct((B1, SI_D, 128, 8, 128), jnp.float32),
    scratch_types=[
        pltpu.VMEM((2, LPW, 128), jnp.int32),
        pltpu.VMEM((2, TPW, DIM), jnp.float32),
        pltpu.VMEM((2, SI_D, LPW, 10, 129), jnp.float32),
        pltpu.SemaphoreType.DMA,
        pltpu.SemaphoreType.DMA,
        pltpu.SemaphoreType.DMA,
    ],
)
def _emb_lookup(tok5, table, out5, idx_v, rows_v, plane_v, sem_g0, sem_g1, sem_o):
    wid = lax.axis_index("s") * NC + lax.axis_index("c")
    li0 = wid * LPW
    iota = lax.iota(jnp.int32, 16)

    def load_idx(b1, slot):
        pltpu.sync_copy(
            tok5.at[b1 // 8, pl.ds(li0, LPW), lax.rem(b1, 8)], idx_v.at[slot]
        )

    def fire_gathers(slot, sem):
        for q in range(LPW):
            pltpu.async_copy(
                table.at[idx_v.at[slot, q]],
                rows_v.at[slot, pl.ds(q * 128, 128)],
                sem,
            )

    def drain_gathers(slot, sem):
        for q in range(LPW):
            pltpu.make_async_copy(
                table.at[idx_v.at[slot, 0]],
                rows_v.at[slot, pl.ds(0, 128)],
                sem,
            ).wait()

    def drain_writebacks(n):
        for _ in range(n):
            pltpu.make_async_copy(
                plane_v.at[0, 0, 0, pl.ds(0, 8), pl.ds(0, 128)],
                out5.at[0, 0, 0],
                sem_o,
            ).wait()

    siv = [(iota + 16 * k) // 8 for k in range(2)]
    drv = [lax.rem(iota + 16 * k, 8) for k in range(2)]

    def transpose_chunk(slot):
        plane = plane_v.at[slot]

        @plsc.parallel_loop(0, TPW, step=1, unroll=4)
        def jloop(j):
            li_s = jnp.full((16,), 0, jnp.int32) + j // 128
            br_s = jnp.full((16,), 0, jnp.int32) + lax.rem(j, 128)
            for k in range(2):
                vec = rows_v[slot, j, pl.ds(k * 16, 16)]
                plsc.store_scatter(plane, [siv[k], li_s, drv[k], br_s], vec)

    def fire_writebacks(slot, b1):
        for si in range(SI_D):
            for li in range(LPW):
                pltpu.async_copy(
                    plane_v.at[slot, si, li, pl.ds(0, 8), pl.ds(0, 128)],
                    out5.at[b1, si, li0 + li],
                    sem_o,
                )

    load_idx(0, 0)
    fire_gathers(0, sem_g0)

    def body(t, carry):
        b1a = 2 * t

        load_idx(b1a + 1, 1)
        fire_gathers(1, sem_g1)

        @pl.when(t >= 1)
        def _():
            drain_writebacks(SI_D * LPW)

        drain_gathers(0, sem_g0)
        transpose_chunk(0)
        fire_writebacks(0, b1a)

        @pl.when(t < T_HALF - 1)
        def _():
            load_idx(b1a + 2, 0)
            fire_gathers(0, sem_g0)

        @pl.when(t >= 1)
        def _():
            drain_writebacks(SI_D * LPW)

        drain_gathers(1, sem_g1)
        transpose_chunk(1)
        fire_writebacks(1, b1a + 1)
        return carry

    lax.fori_loop(0, T_HALF, body, 0)
    drain_writebacks(2 * SI_D * LPW)


def kernel(token_ids, weight):
    tok5 = (
        jnp.asarray(token_ids, jnp.int32)
        .reshape(128, 128, B1 // 8, 8)
        .transpose(2, 0, 3, 1)
    )
    out5 = _emb_lookup(tok5, weight)
    return out5.transpose(2, 4, 0, 1, 3).reshape(B0, B1, DIM)


# speedup vs baseline: 3.7453x; 1.1620x over previous
import functools

import jax
import jax.numpy as jnp
from jax import lax
from jax.experimental import pallas as pl
from jax.experimental.pallas import tpu as pltpu
from jax.experimental.pallas import tpu_sc as plsc

B0, B1, DIM = 16384, 200, 32
NC, NS = 2, 16
NW = NC * NS
TPW = B0 // NW
LPW = TPW // 128
SI_D = DIM // 8
T_HALF = B1 // 2

_mesh = plsc.VectorSubcoreMesh(core_axis_name="c", subcore_axis_name="s")


@functools.partial(
    pl.kernel,
    mesh=_mesh,
    compiler_params=pltpu.CompilerParams(
        use_tc_tiling_on_sc=False, needs_layout_passes=False
    ),
    out_type=jax.ShapeDtypeStruct((B1, SI_D, 128, 8, 128), jnp.float32),
    scratch_types=[
        pltpu.VMEM((2, LPW, 8, 128), jnp.int32),
        pltpu.VMEM((2, TPW, DIM), jnp.float32),
        pltpu.VMEM((2, SI_D, LPW, 10, 129), jnp.float32),
        pltpu.SemaphoreType.DMA,
        pltpu.SemaphoreType.DMA,
        pltpu.SemaphoreType.DMA,
        pltpu.SemaphoreType.DMA,
    ],
)
def _emb_lookup(tok5, table, out5, idx_v, rows_v, plane_v, sem_i, sem_g0, sem_g1, sem_o):
    wid = lax.axis_index("s") * NC + lax.axis_index("c")
    li0 = wid * LPW
    iota = lax.iota(jnp.int32, 16)

    def fire_idx_group(g, gslot):
        pltpu.async_copy(tok5.at[g, pl.ds(li0, LPW)], idx_v.at[gslot], sem_i)

    def drain_idx_group(gslot):
        pltpu.make_async_copy(
            tok5.at[0, pl.ds(li0, LPW)], idx_v.at[gslot], sem_i
        ).wait()

    def fire_gathers(gslot, dr, slot, sem):
        for q in range(LPW):
            pltpu.async_copy(
                table.at[idx_v.at[gslot, q, dr]],
                rows_v.at[slot, pl.ds(q * 128, 128)],
                sem,
            )

    def drain_gathers(slot, sem):
        for q in range(LPW):
            pltpu.make_async_copy(
                table.at[idx_v.at[0, 0, 0]],
                rows_v.at[slot, pl.ds(0, 128)],
                sem,
            ).wait()

    def drain_writebacks(n):
        for _ in range(n):
            pltpu.make_async_copy(
                plane_v.at[0, 0, pl.ds(0, LPW), pl.ds(0, 8), pl.ds(0, 128)],
                out5.at[0, 0, pl.ds(0, LPW)],
                sem_o,
            ).wait()

    siv = [(iota + 16 * k) // 8 for k in range(2)]
    drv = [lax.rem(iota + 16 * k, 8) for k in range(2)]

    def transpose_chunk(slot):
        plane = plane_v.at[slot]

        @plsc.parallel_loop(0, TPW, step=1, unroll=4)
        def jloop(j):
            li_s = jnp.full((16,), 0, jnp.int32) + j // 128
            br_s = jnp.full((16,), 0, jnp.int32) + lax.rem(j, 128)
            for k in range(2):
                vec = rows_v[slot, j, pl.ds(k * 16, 16)]
                plsc.store_scatter(plane, [siv[k], li_s, drv[k], br_s], vec)

    def fire_writebacks(slot, b1):
        for si in range(SI_D):
            pltpu.async_copy(
                plane_v.at[slot, si, pl.ds(0, LPW), pl.ds(0, 8), pl.ds(0, 128)],
                out5.at[b1, si, pl.ds(li0, LPW)],
                sem_o,
            )

    fire_idx_group(0, 0)
    drain_idx_group(0)
    fire_gathers(0, 0, 0, sem_g0)

    def body(t, carry):
        b1a = 2 * t
        g = t // 4
        gslot = lax.rem(g, 2)
        dra = lax.rem(b1a, 8)
        at_group_start = lax.rem(t, 4) == 0

        @pl.when(at_group_start & (t < T_HALF - 4))
        def _():
            fire_idx_group(g + 1, 1 - gslot)

        fire_gathers(gslot, dra + 1, 1, sem_g1)

        @pl.when(t >= 1)
        def _():
            drain_writebacks(SI_D)

        drain_gathers(0, sem_g0)
        transpose_chunk(0)
        fire_writebacks(0, b1a)

        @pl.when(t < T_HALF - 1)
        def _():
            nxt = b1a + 2
            ng = nxt // 8
            ndr = lax.rem(nxt, 8)

            @pl.when(ndr == 0)
            def _():
                drain_idx_group(lax.rem(ng, 2))

            fire_gathers(lax.rem(ng, 2), ndr, 0, sem_g0)

        @pl.when(t >= 1)
        def _():
            drain_writebacks(SI_D)

        drain_gathers(1, sem_g1)
        transpose_chunk(1)
        fire_writebacks(1, b1a + 1)
        return carry

    lax.fori_loop(0, T_HALF, body, 0)
    drain_writebacks(2 * SI_D)


def kernel(token_ids, weight):
    tok5 = (
        jnp.asarray(token_ids, jnp.int32)
        .reshape(128, 128, B1 // 8, 8)
        .transpose(2, 0, 3, 1)
    )
    out5 = _emb_lookup(tok5, weight)
    return out5.transpose(2, 4, 0, 1, 3).reshape(B0, B1, DIM)


# speedup vs baseline: 3.7487x; 1.0009x over previous
import functools

import jax
import jax.numpy as jnp
from jax import lax
from jax.experimental import pallas as pl
from jax.experimental.pallas import tpu as pltpu
from jax.experimental.pallas import tpu_sc as plsc

B0, B1, DIM = 16384, 200, 32
NC, NS = 2, 16
NW = NC * NS
TPW = B0 // NW
LPW = TPW // 128
SI_D = DIM // 8
T_HALF = B1 // 2

_mesh = plsc.VectorSubcoreMesh(core_axis_name="c", subcore_axis_name="s")


@functools.partial(
    pl.kernel,
    mesh=_mesh,
    compiler_params=pltpu.CompilerParams(
        use_tc_tiling_on_sc=False, needs_layout_passes=False, disable_bounds_checks=True
    ),
    out_type=jax.ShapeDtypeStruct((B1, SI_D, 128, 8, 128), jnp.float32),
    scratch_types=[
        pltpu.VMEM((2, LPW, 8, 128), jnp.int32),
        pltpu.VMEM((2, TPW, DIM), jnp.float32),
        pltpu.VMEM((2, SI_D, LPW, 10, 129), jnp.float32),
        pltpu.SemaphoreType.DMA,
        pltpu.SemaphoreType.DMA,
        pltpu.SemaphoreType.DMA,
        pltpu.SemaphoreType.DMA,
    ],
)
def _emb_lookup(tok5, table, out5, idx_v, rows_v, plane_v, sem_i, sem_g0, sem_g1, sem_o):
    wid = lax.axis_index("s") * NC + lax.axis_index("c")
    li0 = wid * LPW
    iota = lax.iota(jnp.int32, 16)

    def fire_idx_group(g, gslot):
        pltpu.async_copy(tok5.at[g, pl.ds(li0, LPW)], idx_v.at[gslot], sem_i)

    def drain_idx_group(gslot):
        pltpu.make_async_copy(
            tok5.at[0, pl.ds(li0, LPW)], idx_v.at[gslot], sem_i
        ).wait()

    def fire_gathers(gslot, dr, slot, sem):
        for q in range(LPW):
            pltpu.async_copy(
                table.at[idx_v.at[gslot, q, dr]],
                rows_v.at[slot, pl.ds(q * 128, 128)],
                sem,
            )

    def drain_gathers(slot, sem):
        for q in range(LPW):
            pltpu.make_async_copy(
                table.at[idx_v.at[0, 0, 0]],
                rows_v.at[slot, pl.ds(0, 128)],
                sem,
            ).wait()

    def drain_writebacks(n):
        for _ in range(n):
            pltpu.make_async_copy(
                plane_v.at[0, 0, pl.ds(0, LPW), pl.ds(0, 8), pl.ds(0, 128)],
                out5.at[0, 0, pl.ds(0, LPW)],
                sem_o,
            ).wait()

    siv = [(iota + 16 * k) // 8 for k in range(2)]
    drv = [lax.rem(iota + 16 * k, 8) for k in range(2)]

    def transpose_chunk(slot):
        plane = plane_v.at[slot]

        @plsc.parallel_loop(0, TPW, step=1, unroll=4)
        def jloop(j):
            li_s = jnp.full((16,), 0, jnp.int32) + j // 128
            br_s = jnp.full((16,), 0, jnp.int32) + lax.rem(j, 128)
            for k in range(2):
                vec = rows_v[slot, j, pl.ds(k * 16, 16)]
                plsc.store_scatter(plane, [siv[k], li_s, drv[k], br_s], vec)

    def fire_writebacks(slot, b1):
        for si in range(SI_D):
            pltpu.async_copy(
                plane_v.at[slot, si, pl.ds(0, LPW), pl.ds(0, 8), pl.ds(0, 128)],
                out5.at[b1, si, pl.ds(li0, LPW)],
                sem_o,
            )

    fire_idx_group(0, 0)
    drain_idx_group(0)
    fire_gathers(0, 0, 0, sem_g0)

    def body(t, carry):
        b1a = 2 * t
        g = t // 4
        gslot = lax.rem(g, 2)
        dra = lax.rem(b1a, 8)
        at_group_start = lax.rem(t, 4) == 0

        @pl.when(at_group_start & (t < T_HALF - 4))
        def _():
            fire_idx_group(g + 1, 1 - gslot)

        fire_gathers(gslot, dra + 1, 1, sem_g1)

        @pl.when(t >= 1)
        def _():
            drain_writebacks(SI_D)

        drain_gathers(0, sem_g0)
        transpose_chunk(0)
        fire_writebacks(0, b1a)

        @pl.when(t < T_HALF - 1)
        def _():
            nxt = b1a + 2
            ng = nxt // 8
            ndr = lax.rem(nxt, 8)

            @pl.when(ndr == 0)
            def _():
                drain_idx_group(lax.rem(ng, 2))

            fire_gathers(lax.rem(ng, 2), ndr, 0, sem_g0)

        @pl.when(t >= 1)
        def _():
            drain_writebacks(SI_D)

        drain_gathers(1, sem_g1)
        transpose_chunk(1)
        fire_writebacks(1, b1a + 1)
        return carry

    lax.fori_loop(0, T_HALF, body, 0)
    drain_writebacks(2 * SI_D)


def kernel(token_ids, weight):
    tok5 = (
        jnp.asarray(token_ids, jnp.int32)
        .reshape(128, 128, B1 // 8, 8)
        .transpose(2, 0, 3, 1)
    )
    out5 = _emb_lookup(tok5, weight)
    return out5.transpose(2, 4, 0, 1, 3).reshape(B0, B1, DIM)


# speedup vs baseline: 3.8042x; 1.0148x over previous
import functools

import jax
import jax.numpy as jnp
from jax import lax
from jax.experimental import pallas as pl
from jax.experimental.pallas import tpu as pltpu
from jax.experimental.pallas import tpu_sc as plsc

B0, B1, DIM = 16384, 200, 32
NC, NS = 2, 16
NW = NC * NS
TPW = B0 // NW
LPW = TPW // 128
SI_D = DIM // 8
T_HALF = B1 // 2

_mesh = plsc.VectorSubcoreMesh(core_axis_name="c", subcore_axis_name="s")


@functools.partial(
    pl.kernel,
    mesh=_mesh,
    compiler_params=pltpu.CompilerParams(
        use_tc_tiling_on_sc=False, needs_layout_passes=False, disable_bounds_checks=True
    ),
    out_type=jax.ShapeDtypeStruct((B1, SI_D, 128, 8, 128), jnp.float32),
    scratch_types=[
        pltpu.VMEM((2, LPW, 8, 128), jnp.int32),
        pltpu.VMEM((2, TPW, DIM), jnp.float32),
        pltpu.VMEM((2, SI_D, LPW, 10, 129), jnp.float32),
        pltpu.SemaphoreType.DMA,
        pltpu.SemaphoreType.DMA,
        pltpu.SemaphoreType.DMA,
        pltpu.SemaphoreType.DMA,
    ],
)
def _emb_lookup(tok5, table, out5, idx_v, rows_v, plane_v, sem_i, sem_g0, sem_g1, sem_o):
    wid = lax.axis_index("s") * NC + lax.axis_index("c")
    li0 = wid * LPW
    iota = lax.iota(jnp.int32, 16)

    def fire_idx_group(g, gslot):
        pltpu.async_copy(tok5.at[g, pl.ds(li0, LPW)], idx_v.at[gslot], sem_i)

    def drain_idx_group(gslot):
        pltpu.make_async_copy(
            tok5.at[0, pl.ds(li0, LPW)], idx_v.at[gslot], sem_i
        ).wait()

    def fire_gathers(gslot, dr, slot, sem):
        for q in range(LPW):
            pltpu.async_copy(
                table.at[idx_v.at[gslot, q, dr]],
                rows_v.at[slot, pl.ds(q * 128, 128)],
                sem,
            )

    def drain_gathers(slot, sem):
        for q in range(LPW):
            pltpu.make_async_copy(
                table.at[idx_v.at[0, 0, 0]],
                rows_v.at[slot, pl.ds(0, 128)],
                sem,
            ).wait()

    def drain_writebacks(n):
        for _ in range(n):
            pltpu.make_async_copy(
                plane_v.at[0, 0, pl.ds(0, LPW), pl.ds(0, 8), pl.ds(0, 128)],
                out5.at[0, 0, pl.ds(0, LPW)],
                sem_o,
            ).wait()

    siv = [(iota + 16 * k) // 8 for k in range(2)]
    drv = [lax.rem(iota + 16 * k, 8) for k in range(2)]

    zeros16 = jnp.full((16,), 0, jnp.int32)
    livs = [zeros16 + li for li in range(LPW)]

    def transpose_chunk(slot):
        plane = plane_v.at[slot]

        @plsc.parallel_loop(0, 128, step=1, unroll=4)
        def brloop(br):
            br_s = zeros16 + br
            for li in range(LPW):
                j = li * 128 + br
                for k in range(2):
                    vec = rows_v[slot, j, pl.ds(k * 16, 16)]
                    plsc.store_scatter(plane, [siv[k], livs[li], drv[k], br_s], vec)

    def fire_writebacks(slot, b1):
        for si in range(SI_D):
            pltpu.async_copy(
                plane_v.at[slot, si, pl.ds(0, LPW), pl.ds(0, 8), pl.ds(0, 128)],
                out5.at[b1, si, pl.ds(li0, LPW)],
                sem_o,
            )

    fire_idx_group(0, 0)
    drain_idx_group(0)
    fire_gathers(0, 0, 0, sem_g0)

    def body(t, carry):
        b1a = 2 * t
        g = t // 4
        gslot = lax.rem(g, 2)
        dra = lax.rem(b1a, 8)
        at_group_start = lax.rem(t, 4) == 0

        @pl.when(at_group_start & (t < T_HALF - 4))
        def _():
            fire_idx_group(g + 1, 1 - gslot)

        fire_gathers(gslot, dra + 1, 1, sem_g1)

        @pl.when(t >= 1)
        def _():
            drain_writebacks(SI_D)

        drain_gathers(0, sem_g0)
        transpose_chunk(0)
        fire_writebacks(0, b1a)

        @pl.when(t < T_HALF - 1)
        def _():
            nxt = b1a + 2
            ng = nxt // 8
            ndr = lax.rem(nxt, 8)

            @pl.when(ndr == 0)
            def _():
                drain_idx_group(lax.rem(ng, 2))

            fire_gathers(lax.rem(ng, 2), ndr, 0, sem_g0)

        @pl.when(t >= 1)
        def _():
            drain_writebacks(SI_D)

        drain_gathers(1, sem_g1)
        transpose_chunk(1)
        fire_writebacks(1, b1a + 1)
        return carry

    lax.fori_loop(0, T_HALF, body, 0)
    drain_writebacks(2 * SI_D)


def kernel(token_ids, weight):
    tok5 = (
        jnp.asarray(token_ids, jnp.int32)
        .reshape(128, 128, B1 // 8, 8)
        .transpose(2, 0, 3, 1)
    )
    out5 = _emb_lookup(tok5, weight)
    return out5.transpose(2, 4, 0, 1, 3).reshape(B0, B1, DIM)
